# baseline (device time: 34385 ns/iter reference)
import jax
import jax.numpy as jnp
import numpy as np
from jax import lax
from jax.experimental import pallas as pl
from jax.experimental.pallas import tpu as pltpu

N_DEV = 32
HOPS = 4
F = (4, 3, 4, 3)
N_STREAMS = 4
SUBS = 2

_LOGICAL_COORDS = []
for _z in range(4):
    for _y in range(4):
        _xs = (0, 1) if _y % 2 == 0 else (1, 0)
        for _x in _xs:
            _LOGICAL_COORDS.append((_x, _y, _z))
_COORD_TO_LOGICAL = {c: i for i, c in enumerate(_LOGICAL_COORDS)}

_P = []
for _y in range(4):
    _zs = range(4) if _y % 2 == 0 else range(3, -1, -1)
    for _z in _zs:
        _P.append((_y, _z))
_CYCLE = [(0, y, z) for (y, z) in _P] + [(1, y, z) for (y, z) in reversed(_P)]
assert len(_CYCLE) == N_DEV
for _a, _b in zip(_CYCLE, _CYCLE[1:] + _CYCLE[:1]):
    assert sum(abs(i - j) for i, j in zip(_a, _b)) == 1, (_a, _b)

_HAM = np.array([_COORD_TO_LOGICAL[c] for c in _CYCLE], dtype=np.int32)
_IDX = np.empty(N_DEV, dtype=np.int32)
_IDX[_HAM] = np.arange(N_DEV, dtype=np.int32)


def kernel(x, w_mat):
    m_per, k = x.shape
    _, n_per = w_mat.shape
    sub_m = m_per // SUBS

    def body(x_ref, w_ref, ham_ref, idx_ref, out_ref, g_buf,
             seed_send, seed_recv, send_r, recv_r, send_l, recv_l):
        my_pos = lax.axis_index("i")
        my_idx = idx_ref[my_pos]

        def at_idx(off):
            return ham_ref[(my_idx + 64 + off) % N_DEV]

        succ = at_idx(1)
        pred = at_idx(-1)

        peers = [pred, succ, at_idx(8), at_idx(-8), at_idx(16)]
        barrier_sem = pltpu.get_barrier_semaphore()
        for nbr in peers:
            pl.semaphore_signal(
                barrier_sem, inc=1,
                device_id=(nbr,), device_id_type=pl.DeviceIdType.MESH,
            )
        pl.semaphore_wait(barrier_sem, len(peers))

        def desc(origin, dst, send_sem, recv_sem, j, from_x=False):
            rows = pl.ds(j * sub_m, sub_m)
            return pltpu.make_async_remote_copy(
                src_ref=x_ref.at[rows] if from_x else g_buf.at[origin, rows],
                dst_ref=g_buf.at[origin, rows],
                send_sem=send_sem,
                recv_sem=recv_sem,
                device_id=(dst,),
                device_id_type=pl.DeviceIdType.MESH,
            )

        def sx(s, c, j):
            return ((s - 1) * N_STREAMS + c) * SUBS + j

        def seedx(c, j):
            return (c - 1) * SUBS + j

        def org(sign, s, c):
            return at_idx(sign * (s - 1) - 8 * c)

        for c in range(1, N_STREAMS):
            for j in range(SUBS):
                desc(my_pos, at_idx(8 * c), seed_send.at[seedx(c, j)],
                     seed_recv.at[seedx(c, j)], j, from_x=True).start()
        for j in range(SUBS):
            desc(org(-1, 1, 0), succ, send_r.at[sx(1, 0, j)],
                 recv_r.at[sx(1, 0, j)], j, from_x=True).start()
            desc(org(+1, 1, 0), pred, send_l.at[sx(1, 0, j)],
                 recv_l.at[sx(1, 0, j)], j, from_x=True).start()

        g_buf[my_pos] = x_ref[...]

        for c in range(1, N_STREAMS):
            for j in range(SUBS):
                desc(at_idx(-8 * c), succ, seed_send.at[seedx(c, j)],
                     seed_recv.at[seedx(c, j)], j).wait_recv()
                desc(org(-1, 1, c), succ, send_r.at[sx(1, c, j)],
                     recv_r.at[sx(1, c, j)], j).start()
                desc(org(+1, 1, c), pred, send_l.at[sx(1, c, j)],
                     recv_l.at[sx(1, c, j)], j).start()

        for s in range(1, HOPS + 1):
            for c in range(N_STREAMS):
                if F[c] < s:
                    continue
                for j in range(SUBS):
                    desc(at_idx(-(s + 8 * c)), succ, send_r.at[sx(s, c, j)],
                         recv_r.at[sx(s, c, j)], j).wait_recv()
                    if s < F[c]:
                        desc(org(-1, s + 1, c), succ,
                             send_r.at[sx(s + 1, c, j)],
                             recv_r.at[sx(s + 1, c, j)], j).start()
                    desc(at_idx(s - 8 * c), pred, send_l.at[sx(s, c, j)],
                         recv_l.at[sx(s, c, j)], j).wait_recv()
                    if s < F[c]:
                        desc(org(+1, s + 1, c), pred,
                             send_l.at[sx(s + 1, c, j)],
                             recv_l.at[sx(s + 1, c, j)], j).start()

        out_ref[...] = jnp.dot(
            g_buf[...].reshape(N_DEV * m_per, k), w_ref[...],
            preferred_element_type=jnp.float32,
        )

        for c in range(1, N_STREAMS):
            for j in range(SUBS):
                desc(my_pos, at_idx(8 * c), seed_send.at[seedx(c, j)],
                     seed_recv.at[seedx(c, j)], j).wait_send()
        for c in range(N_STREAMS):
            for s in range(1, F[c] + 1):
                for j in range(SUBS):
                    desc(my_pos, succ, send_r.at[sx(s, c, j)],
                         recv_r.at[sx(s, c, j)], j).wait_send()
                    desc(my_pos, pred, send_l.at[sx(s, c, j)],
                         recv_l.at[sx(s, c, j)], j).wait_send()

    n_ring_sems = HOPS * N_STREAMS * SUBS
    return pl.pallas_call(
        body,
        out_shape=jax.ShapeDtypeStruct((N_DEV * m_per, n_per), jnp.float32),
        in_specs=[
            pl.BlockSpec(memory_space=pltpu.VMEM),
            pl.BlockSpec(memory_space=pltpu.VMEM),
            pl.BlockSpec(memory_space=pltpu.SMEM),
            pl.BlockSpec(memory_space=pltpu.SMEM),
        ],
        out_specs=pl.BlockSpec(memory_space=pltpu.VMEM),
        scratch_shapes=[
            pltpu.VMEM((N_DEV, m_per, k), jnp.float32),
            pltpu.SemaphoreType.DMA(((N_STREAMS - 1) * SUBS,)),
            pltpu.SemaphoreType.DMA(((N_STREAMS - 1) * SUBS,)),
            pltpu.SemaphoreType.DMA((n_ring_sems,)),
            pltpu.SemaphoreType.DMA((n_ring_sems,)),
            pltpu.SemaphoreType.DMA((n_ring_sems,)),
            pltpu.SemaphoreType.DMA((n_ring_sems,)),
        ],
        compiler_params=pltpu.CompilerParams(collective_id=0),
    )(x, w_mat, jnp.asarray(_HAM), jnp.asarray(_IDX))


# device time: 32974 ns/iter; 1.0428x vs baseline; 1.0428x over previous
import jax
import jax.numpy as jnp
import numpy as np
from jax import lax
from jax.experimental import pallas as pl
from jax.experimental.pallas import tpu as pltpu

N_DEV = 32
HOPS = 4
F = (4, 3, 4, 3)
N_STREAMS = 4

_LOGICAL_COORDS = []
for _z in range(4):
    for _y in range(4):
        _xs = (0, 1) if _y % 2 == 0 else (1, 0)
        for _x in _xs:
            _LOGICAL_COORDS.append((_x, _y, _z))
_COORD_TO_LOGICAL = {c: i for i, c in enumerate(_LOGICAL_COORDS)}

_P = []
for _y in range(4):
    _zs = range(4) if _y % 2 == 0 else range(3, -1, -1)
    for _z in _zs:
        _P.append((_y, _z))
_CYCLE = [(0, y, z) for (y, z) in _P] + [(1, y, z) for (y, z) in reversed(_P)]
assert len(_CYCLE) == N_DEV
for _a, _b in zip(_CYCLE, _CYCLE[1:] + _CYCLE[:1]):
    assert sum(abs(i - j) for i, j in zip(_a, _b)) == 1, (_a, _b)

_HAM = np.array([_COORD_TO_LOGICAL[c] for c in _CYCLE], dtype=np.int32)
_IDX = np.empty(N_DEV, dtype=np.int32)
_IDX[_HAM] = np.arange(N_DEV, dtype=np.int32)


def kernel(x, w_mat):
    m_per, k = x.shape
    _, n_per = w_mat.shape

    def body(x_ref, w_ref, ham_ref, idx_ref, out_ref, g_buf,
             seed_send, seed_recv, send_r, recv_r, send_l, recv_l):
        my_pos = lax.axis_index("i")
        my_idx = idx_ref[my_pos]

        def at_idx(off):
            return ham_ref[(my_idx + 64 + off) % N_DEV]

        succ = at_idx(1)
        pred = at_idx(-1)

        peers = [pred, succ, at_idx(8), at_idx(-8), at_idx(16)]
        barrier_sem = pltpu.get_barrier_semaphore()
        for nbr in peers:
            pl.semaphore_signal(
                barrier_sem, inc=1,
                device_id=(nbr,), device_id_type=pl.DeviceIdType.MESH,
            )
        pl.semaphore_wait(barrier_sem, len(peers))

        def desc(origin, dst, send_sem, recv_sem, src=None):
            return pltpu.make_async_remote_copy(
                src_ref=g_buf.at[origin] if src is None else src,
                dst_ref=g_buf.at[origin],
                send_sem=send_sem,
                recv_sem=recv_sem,
                device_id=(dst,),
                device_id_type=pl.DeviceIdType.MESH,
            )

        def sx(s, c):
            return (s - 1) * N_STREAMS + c

        def org(sign, s, c):
            return at_idx(sign * (s - 1) - 8 * c)

        for c in range(1, N_STREAMS):
            desc(my_pos, at_idx(8 * c),
                 seed_send.at[c - 1], seed_recv.at[c - 1], src=x_ref).start()
        desc(org(-1, 1, 0), succ,
             send_r.at[sx(1, 0)], recv_r.at[sx(1, 0)], src=x_ref).start()
        desc(org(+1, 1, 0), pred,
             send_l.at[sx(1, 0)], recv_l.at[sx(1, 0)], src=x_ref).start()

        g_buf[my_pos] = x_ref[...]

        for c in range(1, N_STREAMS):
            desc(at_idx(-8 * c), succ,
                 seed_send.at[c - 1], seed_recv.at[c - 1]).wait_recv()
            desc(org(-1, 1, c), succ,
                 send_r.at[sx(1, c)], recv_r.at[sx(1, c)]).start()
            desc(org(+1, 1, c), pred,
                 send_l.at[sx(1, c)], recv_l.at[sx(1, c)]).start()

        for s in range(1, HOPS + 1):
            for c in range(N_STREAMS):
                if F[c] < s:
                    continue
                desc(at_idx(-(s + 8 * c)), succ,
                     send_r.at[sx(s, c)], recv_r.at[sx(s, c)]).wait_recv()
                if s < F[c]:
                    desc(org(-1, s + 1, c), succ,
                         send_r.at[sx(s + 1, c)],
                         recv_r.at[sx(s + 1, c)]).start()
                desc(at_idx(s - 8 * c), pred,
                     send_l.at[sx(s, c)], recv_l.at[sx(s, c)]).wait_recv()
                if s < F[c]:
                    desc(org(+1, s + 1, c), pred,
                         send_l.at[sx(s + 1, c)],
                         recv_l.at[sx(s + 1, c)]).start()

        out_ref[...] = jnp.dot(
            g_buf[...].reshape(N_DEV * m_per, k), w_ref[...],
            preferred_element_type=jnp.float32,
        )

        for c in range(1, N_STREAMS):
            desc(my_pos, at_idx(8 * c),
                 seed_send.at[c - 1], seed_recv.at[c - 1]).wait_send()
        for c in range(N_STREAMS):
            for s in range(1, F[c] + 1):
                desc(my_pos, succ,
                     send_r.at[sx(s, c)], recv_r.at[sx(s, c)]).wait_send()
                desc(my_pos, pred,
                     send_l.at[sx(s, c)], recv_l.at[sx(s, c)]).wait_send()

    return pl.pallas_call(
        body,
        out_shape=jax.ShapeDtypeStruct((N_DEV * m_per, n_per), jnp.float32),
        in_specs=[
            pl.BlockSpec(memory_space=pltpu.VMEM),
            pl.BlockSpec(memory_space=pltpu.VMEM),
            pl.BlockSpec(memory_space=pltpu.SMEM),
            pl.BlockSpec(memory_space=pltpu.SMEM),
        ],
        out_specs=pl.BlockSpec(memory_space=pltpu.VMEM),
        scratch_shapes=[
            pltpu.VMEM((N_DEV, m_per, k), jnp.float32),
            pltpu.SemaphoreType.DMA((N_STREAMS - 1,)),
            pltpu.SemaphoreType.DMA((N_STREAMS - 1,)),
            pltpu.SemaphoreType.DMA((HOPS * N_STREAMS,)),
            pltpu.SemaphoreType.DMA((HOPS * N_STREAMS,)),
            pltpu.SemaphoreType.DMA((HOPS * N_STREAMS,)),
            pltpu.SemaphoreType.DMA((HOPS * N_STREAMS,)),
        ],
        compiler_params=pltpu.CompilerParams(collective_id=0),
    )(x, w_mat, jnp.asarray(_HAM), jnp.asarray(_IDX))
